# baseline (speedup 1.0000x reference)
import jax
import jax.numpy as jnp
from jax import lax
from jax.experimental import pallas as pl
from jax.experimental.pallas import tpu as pltpu

_EPS = 1e-5
_F32 = jnp.float32


def _conv_rows(x, wbig, kh, kw, cin, cout, h_out, w_out, p_scr=None,
               g_scr=None):
    h_g = h_out + kh - 1
    if p_scr is not None:
        for t in range(kw):
            p_scr[:h_g, :, t * cin:(t + 1) * cin] = x[:h_g, t:t + w_out, :]
        patch = p_scr[:h_g]
    else:
        patch = jnp.concatenate(
            [x[:h_g, t:t + w_out, :] for t in range(kw)], axis=-1)
    g = jnp.dot(patch.reshape(h_g * w_out, kw * cin), wbig,
                preferred_element_type=_F32)
    if g_scr is not None:
        g_scr[:h_g * w_out] = g
        g = g_scr[:h_g * w_out]
    g = g.reshape(h_g, w_out, kh * cout)
    acc = g[0:h_out, :, 0:cout]
    for u in range(1, kh):
        acc = acc + g[u:u + h_out, :, u * cout:(u + 1) * cout]
    return acc


def _kernel_a0(x_ref, m_ref, b1_ref, out_ref, p_scr):
    x2 = x_ref[0]
    b1t = b1_ref[...]
    for ci, w0 in enumerate((0, 33, 66, 99)):
        p_scr[...] = jnp.concatenate(
            [x2[kh:kh + 132, 3 * w0:3 * w0 + 129] for kh in range(11)],
            axis=1)
        acc = jnp.dot(p_scr[...], m_ref[...], preferred_element_type=_F32)
        out_ref[0, :, w0 * 32:w0 * 32 + 1056] = acc + b1t


def _kernel_a1(c1_ref, rs_ref, rt_ref, wr_ref, cb_ref, s2_ref, t2_ref,
               w2_ref, b2_ref, w3_ref, b3_ref, out_ref,
               band_scr, h2_scr, o2_scr, p2_scr, g2_scr, p3_scr, g3_scr):
    s2 = s2_ref[...]
    t2 = t2_ref[...]
    for i in range(8):
        hs = 18 if i < 7 else 6
        h0 = 18 * i
        for j in range(8):
            ws = 18 if j < 7 else 6
            w0 = 18 * j
            creg = c1_ref[0, h0:h0 + hs, w0:w0 + ws, :]
            yreg = jnp.maximum(creg * rs_ref[i, j] + rt_ref[i, j], 0.0)
            zrow = jnp.zeros((hs, 1, 32), dtype=_F32)
            ypad = jnp.concatenate([zrow, yreg, zrow], axis=1)
            zcol = jnp.zeros((1, ws + 2, 32), dtype=_F32)
            ypad = jnp.concatenate([zcol, ypad, zcol], axis=0)
            conv = _conv_rows(ypad, wr_ref[i, j], 3, 3, 32, 32, hs, ws)
            band_scr[:hs, w0:w0 + ws, :] = conv + cb_ref[i, j] + creg
        v = jnp.maximum(band_scr[:hs], 0.0)
        v = v.reshape(hs // 2, 2, 132, 32)
        v = jnp.maximum(v[:, 0], v[:, 1])
        v = v.reshape(hs // 2, 66, 2, 32)
        v = jnp.maximum(v[:, :, 0, :], v[:, :, 1, :])
        h2_scr[h0 // 2:h0 // 2 + hs // 2] = v * s2 + t2

    b2 = b2_ref[...]
    for r0 in (0, 15, 30, 45):
        n_out = min(15, 59 - r0)
        acc = _conv_rows(h2_scr[r0:r0 + n_out + 7], w2_ref[...],
                         8, 8, 32, 16, n_out, 59, p2_scr, g2_scr)
        o2_scr[r0:r0 + n_out] = jnp.maximum(acc + b2, 0.0)

    acc = _conv_rows(o2_scr[...], w3_ref[...], 8, 8, 16, 16, 52, 52,
                     p3_scr, g3_scr)
    out_ref[0] = jnp.maximum(acc + b3_ref[...], 0.0)


def _kernel_b(xp_ref, w4_ref, b4_ref, w5_ref, b5_ref, out_ref):
    nb = xp_ref.shape[0]
    for b in range(nb):
        acc = None
        for pq in range(4):
            xph = xp_ref[b, pq]
            c = _conv_rows(xph, w4_ref[pq], 3, 3, 16, 16, 24, 24)
            acc = c if acc is None else acc + c
        h4 = jnp.maximum(acc + b4_ref[...], 0.0)
        o5 = _conv_rows(h4, w5_ref[...], 5, 5, 16, 16, 20, 20)
        o5 = jnp.maximum(o5 + b5_ref[...], 0.0)
        out_ref[b] = o5.reshape(400, 16)


def _kernel_c(x_ref, w_ref, b_ref, out_ref):
    out_ref[...] = jnp.maximum(
        jnp.dot(x_ref[...], w_ref[...], preferred_element_type=_F32)
        + b_ref[...], 0.0)


def _kernel_d(x_ref, w2_ref, b2_ref, w3_ref, b3_ref, out_ref):
    h = jnp.maximum(
        jnp.dot(x_ref[...], w2_ref[...], preferred_element_type=_F32)
        + b2_ref[...], 0.0)
    out_ref[...] = (jnp.dot(h, w3_ref[...], preferred_element_type=_F32)
                    + b3_ref[...])


def _full(shape):
    zeros = (0,) * len(shape)
    return pl.BlockSpec(shape, lambda *_: zeros)


@jax.jit
def kernel(x, conv1_w, conv1_b, rg_g, rg_b, rg_m, rg_v, rg_w, rg_cb,
           bn2_g, bn2_b, bn2_m, bn2_v, conv2_w, conv2_b, conv3_w, conv3_b,
           conv4_w, conv4_b, conv5_w, conv5_b, fc1_w, fc1_b, fc2_w, fc2_b,
           fc3_w, fc3_b):
    B = x.shape[0]
    f32 = _F32

    xt = x.transpose(0, 2, 3, 1)
    x2d = xt.reshape(B, 142, 426)

    eye33 = jnp.eye(33, dtype=f32)
    w1t = conv1_w.transpose(2, 3, 1, 0)
    kr = (eye33[None, None, :, None, :, None]
          * w1t[:, :, None, :, None, :]).reshape(11, 11, 99, 1056)
    mband = sum(
        jnp.pad(kr[:, kw], ((0, 0), (3 * kw, 30 - 3 * kw), (0, 0)))
        for kw in range(11)).reshape(11 * 129, 1056)
    b1t = jnp.tile(conv1_b, 33)[None, :]

    rs = rg_g * lax.rsqrt(rg_v + _EPS)
    rt = rg_b - rg_m * rs
    wrbig = rg_w.transpose(0, 1, 5, 3, 4, 2).reshape(8, 8, 96, 96)

    s2 = (bn2_g * lax.rsqrt(bn2_v + _EPS)).reshape(1, 1, 32)
    t2 = (bn2_b - bn2_m * (bn2_g * lax.rsqrt(bn2_v + _EPS))).reshape(1, 1, 32)

    w2big = conv2_w.transpose(3, 1, 2, 0).reshape(256, 128)
    b2 = conv2_b.reshape(1, 1, 16)
    w3big = conv3_w.transpose(3, 1, 2, 0).reshape(128, 128)
    b3 = conv3_b.reshape(1, 1, 16)

    w4ph = (conv4_w.reshape(16, 16, 3, 2, 3, 2)
            .transpose(3, 5, 4, 1, 2, 0).reshape(4, 48, 48))
    b4 = conv4_b.reshape(1, 1, 16)
    w5big = conv5_w.transpose(3, 1, 2, 0).reshape(80, 80)
    b5 = conv5_b.reshape(1, 1, 16)

    fc1p = (fc1_w.reshape(4096, 16, 20, 20).transpose(0, 2, 3, 1)
            .reshape(4096, 6400).T)
    fb1 = fc1_b.reshape(1, 4096)
    fc2p = fc2_w.T
    fb2 = fc2_b.reshape(1, 2048)
    fc3p = fc3_w.T
    fb3 = fc3_b.reshape(1, 12)

    c1_2d = pl.pallas_call(
        _kernel_a0,
        grid=(B,),
        in_specs=[
            pl.BlockSpec((1, 142, 426), lambda b: (b, 0, 0)),
            _full((1419, 1056)), _full((1, 1056)),
        ],
        out_specs=pl.BlockSpec((1, 132, 4224), lambda b: (b, 0, 0)),
        out_shape=jax.ShapeDtypeStruct((B, 132, 4224), f32),
        scratch_shapes=[
            pltpu.VMEM((132, 1419), f32),
        ],
        compiler_params=pltpu.CompilerParams(
            dimension_semantics=("parallel",),
            vmem_limit_bytes=100 * 1024 * 1024,
        ),
    )(x2d, mband, b1t)
    c1 = c1_2d.reshape(B, 132, 132, 32)

    out3 = pl.pallas_call(
        _kernel_a1,
        grid=(B,),
        in_specs=[
            pl.BlockSpec((1, 132, 132, 32), lambda b: (b, 0, 0, 0)),
            _full((8, 8, 32)), _full((8, 8, 32)),
            _full((8, 8, 96, 96)), _full((8, 8, 32)),
            _full((1, 1, 32)), _full((1, 1, 32)),
            _full((256, 128)), _full((1, 1, 16)),
            _full((128, 128)), _full((1, 1, 16)),
        ],
        out_specs=pl.BlockSpec((1, 52, 52, 16), lambda b: (b, 0, 0, 0)),
        out_shape=jax.ShapeDtypeStruct((B, 52, 52, 16), f32),
        scratch_shapes=[
            pltpu.VMEM((18, 132, 32), f32),
            pltpu.VMEM((66, 66, 32), f32),
            pltpu.VMEM((59, 59, 16), f32),
            pltpu.VMEM((22, 59, 256), f32),
            pltpu.VMEM((1298, 128), f32),
            pltpu.VMEM((59, 52, 128), f32),
            pltpu.VMEM((3068, 128), f32),
        ],
        compiler_params=pltpu.CompilerParams(
            dimension_semantics=("parallel",),
            vmem_limit_bytes=100 * 1024 * 1024,
        ),
    )(c1, rs, rt, wrbig, rg_cb, s2, t2, w2big, b2, w3big, b3)

    xph = jnp.stack([out3[:, p::2, q::2, :]
                     for p in range(2) for q in range(2)], axis=1)

    BB = 4 if B % 4 == 0 else 1
    flat = pl.pallas_call(
        _kernel_b,
        grid=(B // BB,),
        in_specs=[
            pl.BlockSpec((BB, 4, 26, 26, 16), lambda b: (b, 0, 0, 0, 0)),
            _full((4, 48, 48)), _full((1, 1, 16)),
            _full((80, 80)), _full((1, 1, 16)),
        ],
        out_specs=pl.BlockSpec((BB, 400, 16), lambda b: (b, 0, 0)),
        out_shape=jax.ShapeDtypeStruct((B, 400, 16), f32),
        compiler_params=pltpu.CompilerParams(
            dimension_semantics=("parallel",),
            vmem_limit_bytes=64 * 1024 * 1024,
        ),
    )(xph, w4ph, b4, w5big, b5)
    flat = flat.reshape(B, 6400)

    NC = 8
    h1 = pl.pallas_call(
        _kernel_c,
        grid=(NC,),
        in_specs=[
            pl.BlockSpec((B, 6400), lambda n: (0, 0)),
            pl.BlockSpec((6400, 4096 // NC), lambda n: (0, n)),
            pl.BlockSpec((1, 4096 // NC), lambda n: (0, n)),
        ],
        out_specs=pl.BlockSpec((B, 4096 // NC), lambda n: (0, n)),
        out_shape=jax.ShapeDtypeStruct((B, 4096), f32),
        compiler_params=pltpu.CompilerParams(
            dimension_semantics=("parallel",),
            vmem_limit_bytes=100 * 1024 * 1024,
        ),
    )(flat, fc1p, fb1)

    out = pl.pallas_call(
        _kernel_d,
        grid=(1,),
        in_specs=[
            _full((B, 4096)), _full((4096, 2048)), _full((1, 2048)),
            _full((2048, 12)), _full((1, 12)),
        ],
        out_specs=_full((B, 12)),
        out_shape=jax.ShapeDtypeStruct((B, 12), f32),
        compiler_params=pltpu.CompilerParams(
            vmem_limit_bytes=100 * 1024 * 1024,
        ),
    )(h1, fc2p, fb2, fc3p, fb3)
    return out
